# Initial kernel scaffold; baseline (speedup 1.0000x reference)
import jax
import jax.numpy as jnp
from jax.experimental import pallas as pl


def kernel(coords, torsions):
    raise NotImplementedError("write your pallas kernel here")


# speedup vs baseline: 10.7556x; 10.7556x over previous
import functools

import jax
import jax.numpy as jnp
from jax import lax
from jax.experimental import pallas as pl
from jax.experimental.pallas import tpu as pltpu
from jax.experimental.pallas import tpu_sc as plsc

N_ATOMS = 500000
N_TORSIONS = 2000000
L = 16
C = 800
NCHUNKS = N_TORSIONS // C
NC = 2
NS = 16
NW = NC * NS
ITERS_PER_W = -(-NCHUNKS // NW)

_PI = 3.14159265358979


def _rsqrt(y):
    i = plsc.bitcast(y, jnp.int32)
    i = jnp.int32(0x5F3759DF) - (i >> 1)
    r = plsc.bitcast(i, jnp.float32)
    for _ in range(3):
        r = r * (1.5 - 0.5 * y * r * r)
    return r


def _acos(x):
    t = jnp.abs(x)
    y = 1.0 - t
    s = y * _rsqrt(jnp.maximum(y, 1e-30))
    p = -0.0012624911
    for a in (0.0066700901, -0.0170881256, 0.0308918810, -0.0501743046,
              0.0889789874, -0.2145988016, 1.5707963050):
        p = p * t + a
    r = s * p
    return jnp.where(x >= 0, r, _PI - r)


def _torsion_sc_kernel(tors_hbm, coords_hbm, out_hbm,
                       table_sh, idx_v, gidx_v, rows_v, out_v, sem):
    cid = lax.axis_index("c")
    sid = lax.axis_index("s")
    wid = sid * NC + cid

    NPIECE = 6000
    NPIECES = (3 * N_ATOMS) // NPIECE

    def stage(q0, carry):
        q = sid + q0 * NS

        @pl.when(q < NPIECES)
        def _():
            pltpu.sync_copy(coords_hbm.at[pl.ds(q * NPIECE, NPIECE)], rows_v.at[pl.ds(0, NPIECE)])
            pltpu.sync_copy(rows_v.at[pl.ds(0, NPIECE)], table_sh.at[pl.ds(q * NPIECE, NPIECE)])

        return carry

    lax.fori_loop(0, -(-NPIECES // NS), stage, 0)
    plsc.subcore_barrier()

    iota = lax.iota(jnp.int32, L)
    iota4 = iota * 4

    def do_chunk(g):
        pltpu.sync_copy(tors_hbm.at[pl.ds(g * (4 * C), 4 * C)], idx_v)

        def build(b, carry):
            base = b * (4 * L)
            for p in range(4):
                ap = plsc.load_gather(idx_v, [base + iota4 + p])
                m = ap * 3
                for c in range(3):
                    gidx_v[pl.ds((3 * p + c) * C + b * L, L)] = m + c
            return carry

        lax.fori_loop(0, C // L, build, 0)

        pltpu.async_copy(table_sh.at[gidx_v], rows_v, sem).wait()

        def body(b, carry):
            o = b * L
            r = [[rows_v[pl.ds((3 * p + c) * C + o, L)] for c in range(3)]
                 for p in range(4)]
            b1 = [r[1][c] - r[0][c] for c in range(3)]
            b2 = [r[2][c] - r[1][c] for c in range(3)]
            b3 = [r[3][c] - r[2][c] for c in range(3)]
            n1 = [b1[1] * b2[2] - b1[2] * b2[1],
                  b1[2] * b2[0] - b1[0] * b2[2],
                  b1[0] * b2[1] - b1[1] * b2[0]]
            n2 = [b2[1] * b3[2] - b2[2] * b3[1],
                  b2[2] * b3[0] - b2[0] * b3[2],
                  b2[0] * b3[1] - b2[1] * b3[0]]
            d = n1[0] * n2[0] + n1[1] * n2[1] + n1[2] * n2[2]
            n1sq = n1[0] * n1[0] + n1[1] * n1[1] + n1[2] * n1[2]
            n2sq = n2[0] * n2[0] + n2[1] * n2[1] + n2[2] * n2[2]
            sdot = n1[0] * b3[0] + n1[1] * b3[1] + n1[2] * b3[2]
            denom = n1sq * n2sq
            sq = denom * _rsqrt(jnp.maximum(denom, 1e-35))
            cos_raw = d / sq
            cos_cl = jnp.minimum(jnp.maximum(cos_raw, -0.999999999), 0.99999999)
            is_nan = cos_raw != cos_raw
            cos = jnp.where(is_nan, cos_raw, cos_cl)
            phi = _acos(cos)
            phi = jnp.where(is_nan, cos, phi)
            phi = jnp.where(sdot > 0, phi, -phi)
            out_v[pl.ds(o, L)] = phi
            return carry

        lax.fori_loop(0, C // L, body, 0)
        pltpu.sync_copy(out_v, out_hbm.at[pl.ds(g * C, C)])

    def chunk_loop(t, carry):
        g = wid + t * NW

        @pl.when(g < NCHUNKS)
        def _():
            do_chunk(g)

        return carry

    lax.fori_loop(0, ITERS_PER_W, chunk_loop, 0)


def kernel(coords, torsions):
    coords_flat = coords.reshape(-1)
    tors_flat = torsions.reshape(-1)

    mesh = plsc.VectorSubcoreMesh(core_axis_name="c", subcore_axis_name="s")
    run = functools.partial(
        pl.kernel,
        mesh=mesh,
        compiler_params=pltpu.CompilerParams(needs_layout_passes=False),
        out_type=jax.ShapeDtypeStruct((N_TORSIONS,), jnp.float32),
        scratch_types=[
            pltpu.VMEM_SHARED((3 * N_ATOMS,), jnp.float32),
            pltpu.VMEM((4 * C,), jnp.int32),
            pltpu.VMEM((12 * C,), jnp.int32),
            pltpu.VMEM((12 * C,), jnp.float32),
            pltpu.VMEM((C,), jnp.float32),
            pltpu.SemaphoreType.DMA,
        ],
    )(_torsion_sc_kernel)
    return run(tors_flat, coords_flat)


# speedup vs baseline: 38.1782x; 3.5496x over previous
import functools

import jax
import jax.numpy as jnp
from jax import lax
from jax.experimental import pallas as pl
from jax.experimental.pallas import tpu as pltpu
from jax.experimental.pallas import tpu_sc as plsc

N_ATOMS = 500000
N_TORSIONS = 2000000
L = 16
C = 800
NCHUNKS = N_TORSIONS // C
NC = 2
NS = 16
NW = NC * NS
ITERS_PER_W = -(-NCHUNKS // NW)
NPIECE = 6000
NPIECES = (3 * N_ATOMS) // NPIECE

_PI = 3.14159265358979


def _rsqrt(y):
    i = plsc.bitcast(y, jnp.int32)
    i = jnp.int32(0x5F3759DF) - (i >> 1)
    r = plsc.bitcast(i, jnp.float32)
    for _ in range(3):
        r = r * (1.5 - 0.5 * y * r * r)
    return r


def _acos(x):
    t = jnp.abs(x)
    y = 1.0 - t
    s = y * _rsqrt(jnp.maximum(y, 1e-30))
    p = -0.0012624911
    for a in (0.0066700901, -0.0170881256, 0.0308918810, -0.0501743046,
              0.0889789874, -0.2145988016, 1.5707963050):
        p = p * t + a
    r = s * p
    return jnp.where(x >= 0, r, _PI - r)


def _torsion_sc_kernel(tors_hbm, coords_hbm, out_hbm,
                       table_sh, idx_v, gidx_v, rows_v, out_v, sem):
    cid = lax.axis_index("c")
    sid = lax.axis_index("s")
    wid = sid * NC + cid

    def stage(q0, carry):
        q = sid + q0 * NS

        @pl.when(q < NPIECES)
        def _():
            pltpu.sync_copy(coords_hbm.at[pl.ds(q * NPIECE, NPIECE)],
                            rows_v.at[pl.ds(0, NPIECE)])
            pltpu.sync_copy(rows_v.at[pl.ds(0, NPIECE)],
                            table_sh.at[pl.ds(q * NPIECE, NPIECE)])

        return carry

    lax.fori_loop(0, -(-NPIECES // NS), stage, 0)
    plsc.subcore_barrier()

    def do_chunk(g):
        for p in range(4):
            pltpu.sync_copy(tors_hbm.at[pl.ds(p * N_TORSIONS + g * C, C)],
                            idx_v.at[pl.ds(p * C, C)])

        def build(b, carry):
            o = b * L
            for p in range(4):
                ap = idx_v[pl.ds(p * C + o, L)]
                for c in range(3):
                    gidx_v[pl.ds((3 * p + c) * C + o, L)] = ap + c * N_ATOMS
            return carry

        lax.fori_loop(0, C // L, build, 0)

        pltpu.async_copy(table_sh.at[gidx_v], rows_v, sem).wait()

        def body(b, carry):
            o = b * L
            r = [[rows_v[pl.ds((3 * p + c) * C + o, L)] for c in range(3)]
                 for p in range(4)]
            b1 = [r[1][c] - r[0][c] for c in range(3)]
            b2 = [r[2][c] - r[1][c] for c in range(3)]
            b3 = [r[3][c] - r[2][c] for c in range(3)]
            n1 = [b1[1] * b2[2] - b1[2] * b2[1],
                  b1[2] * b2[0] - b1[0] * b2[2],
                  b1[0] * b2[1] - b1[1] * b2[0]]
            n2 = [b2[1] * b3[2] - b2[2] * b3[1],
                  b2[2] * b3[0] - b2[0] * b3[2],
                  b2[0] * b3[1] - b2[1] * b3[0]]
            d = n1[0] * n2[0] + n1[1] * n2[1] + n1[2] * n2[2]
            n1sq = n1[0] * n1[0] + n1[1] * n1[1] + n1[2] * n1[2]
            n2sq = n2[0] * n2[0] + n2[1] * n2[1] + n2[2] * n2[2]
            sdot = n1[0] * b3[0] + n1[1] * b3[1] + n1[2] * b3[2]
            denom = n1sq * n2sq
            sq = denom * _rsqrt(jnp.maximum(denom, 1e-35))
            cos_raw = d / sq
            cos_cl = jnp.minimum(jnp.maximum(cos_raw, -0.999999999), 0.99999999)
            is_nan = cos_raw != cos_raw
            cos = jnp.where(is_nan, cos_raw, cos_cl)
            phi = _acos(cos)
            phi = jnp.where(is_nan, cos, phi)
            phi = jnp.where(sdot > 0, phi, -phi)
            out_v[pl.ds(o, L)] = phi
            return carry

        lax.fori_loop(0, C // L, body, 0)
        pltpu.sync_copy(out_v, out_hbm.at[pl.ds(g * C, C)])

    def chunk_loop(t, carry):
        g = wid + t * NW

        @pl.when(g < NCHUNKS)
        def _():
            do_chunk(g)

        return carry

    lax.fori_loop(0, ITERS_PER_W, chunk_loop, 0)


def kernel(coords, torsions):
    coords_t = coords.T.reshape(-1)
    tors_t = torsions.T.reshape(-1)

    mesh = plsc.VectorSubcoreMesh(core_axis_name="c", subcore_axis_name="s")
    run = functools.partial(
        pl.kernel,
        mesh=mesh,
        compiler_params=pltpu.CompilerParams(needs_layout_passes=False),
        out_type=jax.ShapeDtypeStruct((N_TORSIONS,), jnp.float32),
        scratch_types=[
            pltpu.VMEM_SHARED((3 * N_ATOMS,), jnp.float32),
            pltpu.VMEM((4 * C,), jnp.int32),
            pltpu.VMEM((12 * C,), jnp.int32),
            pltpu.VMEM((12 * C,), jnp.float32),
            pltpu.VMEM((C,), jnp.float32),
            pltpu.SemaphoreType.DMA,
        ],
    )(_torsion_sc_kernel)
    return run(tors_t, coords_t)


# speedup vs baseline: 44.9652x; 1.1778x over previous
import functools

import jax
import jax.numpy as jnp
from jax import lax
from jax.experimental import pallas as pl
from jax.experimental.pallas import tpu as pltpu
from jax.experimental.pallas import tpu_sc as plsc

N_ATOMS = 500000
N_TORSIONS = 2000000
L = 16
C = 800
NCHUNKS = N_TORSIONS // C
NC = 2
NS = 16
NW = NC * NS
ITERS_PER_W = -(-NCHUNKS // NW)
NPIECE = 4000
NPIECES = (3 * N_ATOMS) // NPIECE

_PI = 3.14159265358979


def _rsqrt(y):
    i = plsc.bitcast(y, jnp.int32)
    i = jnp.int32(0x5F3759DF) - (i >> 1)
    r = plsc.bitcast(i, jnp.float32)
    for _ in range(3):
        r = r * (1.5 - 0.5 * y * r * r)
    return r


def _acos(x):
    t = jnp.abs(x)
    y = 1.0 - t
    s = y * _rsqrt(jnp.maximum(y, 1e-30))
    p = -0.0012624911
    for a in (0.0066700901, -0.0170881256, 0.0308918810, -0.0501743046,
              0.0889789874, -0.2145988016, 1.5707963050):
        p = p * t + a
    r = s * p
    return jnp.where(x >= 0, r, _PI - r)


def _torsion_sc_kernel(tors_hbm, coords_hbm, out_hbm,
                       table_sh, idx_v, gidx_v, gidx2_v, rows_v, rows2_v,
                       out_v, sem, sem2):
    cid = lax.axis_index("c")
    sid = lax.axis_index("s")
    wid = sid * NC + cid

    def stage(q0, carry):
        q = sid + q0 * NS

        @pl.when(q < NPIECES)
        def _():
            pltpu.sync_copy(coords_hbm.at[pl.ds(q * NPIECE, NPIECE)],
                            rows_v.at[pl.ds(0, NPIECE)])
            pltpu.sync_copy(rows_v.at[pl.ds(0, NPIECE)],
                            table_sh.at[pl.ds(q * NPIECE, NPIECE)])

        return carry

    lax.fori_loop(0, -(-NPIECES // NS), stage, 0)
    plsc.subcore_barrier()

    def do_chunk(g):
        for p in range(4):
            pltpu.sync_copy(tors_hbm.at[pl.ds(p * N_TORSIONS + g * C, C)],
                            idx_v.at[pl.ds(p * C, C)])

        def build(b, carry):
            o = b * L
            for p in range(4):
                ap = idx_v[pl.ds(p * C + o, L)]
                for c in range(3):
                    q = 3 * p + c
                    if q < 6:
                        gidx_v[pl.ds(q * C + o, L)] = ap + c * N_ATOMS
                    else:
                        gidx2_v[pl.ds((q - 6) * C + o, L)] = ap + c * N_ATOMS
            return carry

        lax.fori_loop(0, C // L, build, 0)

        h1 = pltpu.async_copy(table_sh.at[gidx_v], rows_v, sem)
        h2 = pltpu.async_copy(table_sh.at[gidx2_v], rows2_v, sem2)
        h1.wait()
        h2.wait()

        def body(b, carry):
            o = b * L
            r = [[(rows_v[pl.ds((3 * p + c) * C + o, L)] if 3 * p + c < 6
                   else rows2_v[pl.ds((3 * p + c - 6) * C + o, L)])
                  for c in range(3)] for p in range(4)]
            b1 = [r[1][c] - r[0][c] for c in range(3)]
            b2 = [r[2][c] - r[1][c] for c in range(3)]
            b3 = [r[3][c] - r[2][c] for c in range(3)]
            n1 = [b1[1] * b2[2] - b1[2] * b2[1],
                  b1[2] * b2[0] - b1[0] * b2[2],
                  b1[0] * b2[1] - b1[1] * b2[0]]
            n2 = [b2[1] * b3[2] - b2[2] * b3[1],
                  b2[2] * b3[0] - b2[0] * b3[2],
                  b2[0] * b3[1] - b2[1] * b3[0]]
            d = n1[0] * n2[0] + n1[1] * n2[1] + n1[2] * n2[2]
            n1sq = n1[0] * n1[0] + n1[1] * n1[1] + n1[2] * n1[2]
            n2sq = n2[0] * n2[0] + n2[1] * n2[1] + n2[2] * n2[2]
            sdot = n1[0] * b3[0] + n1[1] * b3[1] + n1[2] * b3[2]
            denom = n1sq * n2sq
            sq = denom * _rsqrt(jnp.maximum(denom, 1e-35))
            cos_raw = d / sq
            cos_cl = jnp.minimum(jnp.maximum(cos_raw, -0.999999999), 0.99999999)
            is_nan = cos_raw != cos_raw
            cos = jnp.where(is_nan, cos_raw, cos_cl)
            phi = _acos(cos)
            phi = jnp.where(is_nan, cos, phi)
            phi = jnp.where(sdot > 0, phi, -phi)
            out_v[pl.ds(o, L)] = phi
            return carry

        lax.fori_loop(0, C // L, body, 0)
        pltpu.sync_copy(out_v, out_hbm.at[pl.ds(g * C, C)])

    def chunk_loop(t, carry):
        g = wid + t * NW

        @pl.when(g < NCHUNKS)
        def _():
            do_chunk(g)

        return carry

    lax.fori_loop(0, ITERS_PER_W, chunk_loop, 0)


def kernel(coords, torsions):
    coords_t = coords.T.reshape(-1)
    tors_t = torsions.T.reshape(-1)

    mesh = plsc.VectorSubcoreMesh(core_axis_name="c", subcore_axis_name="s")
    run = functools.partial(
        pl.kernel,
        mesh=mesh,
        compiler_params=pltpu.CompilerParams(needs_layout_passes=False,
                                             use_tc_tiling_on_sc=False),
        out_type=jax.ShapeDtypeStruct((N_TORSIONS,), jnp.float32),
        scratch_types=[
            pltpu.VMEM_SHARED((3 * N_ATOMS,), jnp.float32),
            pltpu.VMEM((4 * C,), jnp.int32),
            pltpu.VMEM((6 * C,), jnp.int32),
            pltpu.VMEM((6 * C,), jnp.int32),
            pltpu.VMEM((6 * C,), jnp.float32),
            pltpu.VMEM((6 * C,), jnp.float32),
            pltpu.VMEM((C,), jnp.float32),
            pltpu.SemaphoreType.DMA,
            pltpu.SemaphoreType.DMA,
        ],
    )(_torsion_sc_kernel)
    return run(tors_t, coords_t)
